# Initial kernel scaffold; baseline (speedup 1.0000x reference)
import functools

import jax
import jax.numpy as jnp
from jax import lax
from jax.experimental import pallas as pl
from jax.experimental.pallas import tpu as pltpu
from jax.experimental.pallas import tpu_sc as plsc

N_NODES = 10000
N_EDGES = 320000
D = 128

NC = 2
NS = 16
NW = NC * NS
EDGES_PER_W = N_EDGES // NW
BLK = 128
NFULL = EDGES_PER_W // BLK
TAIL = EDGES_PER_W - NFULL * BLK
ROWS_PER_TILE = N_NODES // NS
ZROWS = 125

_mesh = lambda: plsc.VectorSubcoreMesh(core_axis_name="c", subcore_axis_name="s")


def _zero_fill(buf, nrows):
    @pl.loop(0, nrows)
    def _(i):
        @pl.loop(0, D, step=16)
        def _(j):
            buf[i, pl.ds(j, 16)] = jnp.zeros((16,), jnp.float32)


def _sc_message_pass(table, src, dst):
    @functools.partial(
        pl.kernel,
        out_type=jax.ShapeDtypeStruct((NC, N_NODES, D), jnp.float32),
        mesh=_mesh(),
        scratch_types=[
            pltpu.VMEM((BLK,), jnp.int32),
            pltpu.VMEM((BLK,), jnp.int32),
            pltpu.VMEM((BLK, D), jnp.float32),
            pltpu.VMEM((TAIL,), jnp.int32),
            pltpu.VMEM((TAIL,), jnp.int32),
            pltpu.VMEM((TAIL, D), jnp.float32),
            pltpu.VMEM((ZROWS, D), jnp.float32),
            pltpu.VMEM_SHARED((N_NODES, D), jnp.float32),
            pltpu.SemaphoreType.DMA,
        ],
    )
    def k(table_hbm, src_hbm, dst_hbm, out_hbm,
          sidx, didx, rows, tsidx, tdidx, trows, zbuf, acc, sem):
        cid = lax.axis_index("c")
        sid = lax.axis_index("s")
        wid = sid * NC + cid
        base = wid * EDGES_PER_W
        my_row0 = sid * ROWS_PER_TILE

        _zero_fill(zbuf, ZROWS)
        @pl.loop(0, ROWS_PER_TILE, step=ZROWS)
        def _(r):
            pltpu.sync_copy(zbuf, acc.at[pl.ds(my_row0 + r, ZROWS)])
        plsc.subcore_barrier()

        @pl.loop(0, NFULL)
        def _(b):
            off = base + b * BLK
            pltpu.sync_copy(src_hbm.at[pl.ds(off, BLK)], sidx)
            pltpu.sync_copy(dst_hbm.at[pl.ds(off, BLK)], didx)
            pltpu.async_copy(table_hbm.at[sidx], rows, sem).wait()
            pltpu.sync_copy(rows, acc.at[didx], add=True)

        toff = base + NFULL * BLK
        pltpu.sync_copy(src_hbm.at[pl.ds(toff, TAIL)], tsidx)
        pltpu.sync_copy(dst_hbm.at[pl.ds(toff, TAIL)], tdidx)
        pltpu.async_copy(table_hbm.at[tsidx], trows, sem).wait()
        pltpu.sync_copy(trows, acc.at[tdidx], add=True)

        plsc.subcore_barrier()

        pltpu.sync_copy(acc.at[pl.ds(my_row0, ROWS_PER_TILE)],
                        out_hbm.at[cid].at[pl.ds(my_row0, ROWS_PER_TILE)])

    return k(table, src, dst)


def _sc_degrees(src, dst):
    @functools.partial(
        pl.kernel,
        out_type=jax.ShapeDtypeStruct((NC, 2, N_NODES), jnp.float32),
        mesh=_mesh(),
        scratch_types=[
            pltpu.VMEM((BLK,), jnp.int32),
            pltpu.VMEM((BLK,), jnp.int32),
            pltpu.VMEM((TAIL,), jnp.int32),
            pltpu.VMEM((TAIL,), jnp.int32),
            pltpu.VMEM((BLK,), jnp.float32),
            pltpu.VMEM((TAIL,), jnp.float32),
            pltpu.VMEM((ZROWS,), jnp.float32),
            pltpu.VMEM_SHARED((N_NODES,), jnp.float32),
            pltpu.VMEM_SHARED((N_NODES,), jnp.float32),
        ],
    )
    def k(src_hbm, dst_hbm, out_hbm,
          sidx, didx, tsidx, tdidx, ones, tones, zbuf, acc_s, acc_d):
        cid = lax.axis_index("c")
        sid = lax.axis_index("s")
        wid = sid * NC + cid
        base = wid * EDGES_PER_W
        my_row0 = sid * ROWS_PER_TILE

        @pl.loop(0, ZROWS, step=16)
        def _(j):
            zbuf[pl.ds(j, 16)] = jnp.zeros((16,), jnp.float32)
        @pl.loop(0, BLK, step=16)
        def _(j):
            ones[pl.ds(j, 16)] = jnp.ones((16,), jnp.float32)
        tones[pl.ds(0, TAIL)] = jnp.ones((TAIL,), jnp.float32)

        @pl.loop(0, ROWS_PER_TILE, step=ZROWS)
        def _(r):
            pltpu.sync_copy(zbuf, acc_s.at[pl.ds(my_row0 + r, ZROWS)])
            pltpu.sync_copy(zbuf, acc_d.at[pl.ds(my_row0 + r, ZROWS)])
        plsc.subcore_barrier()

        @pl.loop(0, NFULL)
        def _(b):
            off = base + b * BLK
            pltpu.sync_copy(src_hbm.at[pl.ds(off, BLK)], sidx)
            pltpu.sync_copy(dst_hbm.at[pl.ds(off, BLK)], didx)
            pltpu.sync_copy(ones, acc_s.at[sidx], add=True)
            pltpu.sync_copy(ones, acc_d.at[didx], add=True)

        toff = base + NFULL * BLK
        pltpu.sync_copy(src_hbm.at[pl.ds(toff, TAIL)], tsidx)
        pltpu.sync_copy(dst_hbm.at[pl.ds(toff, TAIL)], tdidx)
        pltpu.sync_copy(tones, acc_s.at[tsidx], add=True)
        pltpu.sync_copy(tones, acc_d.at[tdidx], add=True)

        plsc.subcore_barrier()

        pltpu.sync_copy(acc_s.at[pl.ds(my_row0, ROWS_PER_TILE)],
                        out_hbm.at[cid].at[0].at[pl.ds(my_row0, ROWS_PER_TILE)])
        pltpu.sync_copy(acc_d.at[pl.ds(my_row0, ROWS_PER_TILE)],
                        out_hbm.at[cid].at[1].at[pl.ds(my_row0, ROWS_PER_TILE)])

    return k(src, dst)


def _norms(d_ref):
    deg_out = d_ref[:, 0:1] + d_ref[:, 2:3]
    deg_in = d_ref[:, 1:2] + d_ref[:, 3:4]
    return (lax.rsqrt(jnp.maximum(deg_out, 1.0)),
            lax.rsqrt(jnp.maximum(deg_in, 1.0)))


def _tc_matmul(x, w):
    def body(x_ref, w_ref, o_ref):
        o_ref[...] = jnp.dot(x_ref[...], w_ref[...],
                             preferred_element_type=jnp.float32)
    return pl.pallas_call(
        body, out_shape=jax.ShapeDtypeStruct((x.shape[0], w.shape[1]), jnp.float32),
    )(x, w)


def _tc_scale_src(xw, degp_t):
    def body(xw_ref, d_ref, o_ref):
        ns, _ = _norms(d_ref)
        o_ref[...] = xw_ref[...] * ns
    return pl.pallas_call(
        body, out_shape=jax.ShapeDtypeStruct(xw.shape, jnp.float32),
    )(xw, degp_t)


def _tc_mid(partials, degp_t, b1, w2):
    def body(p_ref, d_ref, b_ref, w_ref, o_ref):
        ns, nd = _norms(d_ref)
        u = (p_ref[0] + p_ref[1]) * nd + b_ref[...]
        u = jnp.maximum(u, 0.0)
        o_ref[...] = jnp.dot(u, w_ref[...],
                             preferred_element_type=jnp.float32) * ns
    return pl.pallas_call(
        body, out_shape=jax.ShapeDtypeStruct((N_NODES, D), jnp.float32),
    )(partials, degp_t, b1, w2)


def _tc_final(partials, degp_t, b2):
    def body(p_ref, d_ref, b_ref, o_ref):
        _, nd = _norms(d_ref)
        o_ref[...] = (p_ref[0] + p_ref[1]) * nd + b_ref[...]
    return pl.pallas_call(
        body, out_shape=jax.ShapeDtypeStruct((N_NODES, D), jnp.float32),
    )(partials, degp_t, b2)


def kernel(features, edge_index, W1, b1, W2, b2):
    src = edge_index[0].astype(jnp.int32)
    dst = edge_index[1].astype(jnp.int32)
    b1r = b1.reshape(1, D)
    b2r = b2.reshape(1, D)

    degp = _sc_degrees(src, dst)
    degp_t = jnp.transpose(degp.reshape(NC * 2, N_NODES))

    xw1 = _tc_matmul(features, W1)
    h1 = _tc_scale_src(xw1, degp_t)
    p1 = _sc_message_pass(h1, src, dst)
    h2 = _tc_mid(p1, degp_t, b1r, W2)
    p2 = _sc_message_pass(h2, src, dst)
    return _tc_final(p2, degp_t, b2r)


# speedup vs baseline: 6.2869x; 6.2869x over previous
import functools

import jax
import jax.numpy as jnp
from jax import lax
from jax.experimental import pallas as pl
from jax.experimental.pallas import tpu as pltpu
from jax.experimental.pallas import tpu_sc as plsc

N_NODES = 10000
N_EDGES = 320000
D = 128

NC = 2
NS = 16
NW = NC * NS
EDGES_PER_W = N_EDGES // NW
BLK = 128
NFULL = EDGES_PER_W // BLK
TAIL = EDGES_PER_W - NFULL * BLK
N_PAD = 10240
ROWS_PER_TILE = N_PAD // NS
ZROWS = 128

_mesh = lambda: plsc.VectorSubcoreMesh(core_axis_name="c", subcore_axis_name="s")


def _sc_message_pass(table, src, dst):
    @functools.partial(
        pl.kernel,
        out_type=jax.ShapeDtypeStruct((NC, N_PAD, D), jnp.float32),
        mesh=_mesh(),
        scratch_types=[
            pltpu.VMEM((BLK,), jnp.int32),
            pltpu.VMEM((BLK,), jnp.int32),
            pltpu.VMEM((BLK, D), jnp.float32),
            pltpu.VMEM((TAIL,), jnp.int32),
            pltpu.VMEM((TAIL,), jnp.int32),
            pltpu.VMEM((TAIL, D), jnp.float32),
            pltpu.VMEM((ZROWS, D), jnp.float32),
            pltpu.VMEM_SHARED((N_PAD, D), jnp.float32),
            pltpu.SemaphoreType.DMA,
        ],
    )
    def k(table_hbm, src_hbm, dst_hbm, out_hbm,
          sidx, didx, rows, tsidx, tdidx, trows, zbuf, acc, sem):
        cid = lax.axis_index("c")
        sid = lax.axis_index("s")
        wid = sid * NC + cid
        base = wid * EDGES_PER_W
        my_row0 = sid * ROWS_PER_TILE

        @pl.loop(0, ZROWS)
        def _(i):
            @pl.loop(0, D, step=16)
            def _(j):
                zbuf[i, pl.ds(j, 16)] = jnp.zeros((16,), jnp.float32)
        @pl.loop(0, ROWS_PER_TILE, step=ZROWS)
        def _(r):
            pltpu.sync_copy(zbuf, acc.at[pl.ds(my_row0 + r, ZROWS)])
        plsc.subcore_barrier()

        @pl.loop(0, NFULL)
        def _(b):
            off = base + b * BLK
            pltpu.sync_copy(src_hbm.at[pl.ds(off, BLK)], sidx)
            pltpu.sync_copy(dst_hbm.at[pl.ds(off, BLK)], didx)
            pltpu.async_copy(table_hbm.at[sidx], rows, sem).wait()
            pltpu.sync_copy(rows, acc.at[didx], add=True)

        toff = base + NFULL * BLK
        pltpu.sync_copy(src_hbm.at[pl.ds(toff, TAIL)], tsidx)
        pltpu.sync_copy(dst_hbm.at[pl.ds(toff, TAIL)], tdidx)
        pltpu.async_copy(table_hbm.at[tsidx], trows, sem).wait()
        pltpu.sync_copy(trows, acc.at[tdidx], add=True)

        plsc.subcore_barrier()

        pltpu.sync_copy(acc.at[pl.ds(my_row0, ROWS_PER_TILE)],
                        out_hbm.at[cid].at[pl.ds(my_row0, ROWS_PER_TILE)])

    return k(table, src, dst)


def _sc_degrees(src, dst):
    @functools.partial(
        pl.kernel,
        out_type=jax.ShapeDtypeStruct((NC, 2, N_PAD), jnp.float32),
        mesh=_mesh(),
        scratch_types=[
            pltpu.VMEM((BLK,), jnp.int32),
            pltpu.VMEM((BLK,), jnp.int32),
            pltpu.VMEM((TAIL,), jnp.int32),
            pltpu.VMEM((TAIL,), jnp.int32),
            pltpu.VMEM((BLK,), jnp.float32),
            pltpu.VMEM((TAIL,), jnp.float32),
            pltpu.VMEM((ROWS_PER_TILE,), jnp.float32),
            pltpu.VMEM_SHARED((N_PAD,), jnp.float32),
            pltpu.VMEM_SHARED((N_PAD,), jnp.float32),
        ],
    )
    def k(src_hbm, dst_hbm, out_hbm,
          sidx, didx, tsidx, tdidx, ones, tones, zbuf, acc_s, acc_d):
        cid = lax.axis_index("c")
        sid = lax.axis_index("s")
        wid = sid * NC + cid
        base = wid * EDGES_PER_W
        my_row0 = sid * ROWS_PER_TILE

        @pl.loop(0, ROWS_PER_TILE, step=16)
        def _(j):
            zbuf[pl.ds(j, 16)] = jnp.zeros((16,), jnp.float32)
        @pl.loop(0, BLK, step=16)
        def _(j):
            ones[pl.ds(j, 16)] = jnp.ones((16,), jnp.float32)
        tones[pl.ds(0, TAIL)] = jnp.ones((TAIL,), jnp.float32)

        pltpu.sync_copy(zbuf, acc_s.at[pl.ds(my_row0, ROWS_PER_TILE)])
        pltpu.sync_copy(zbuf, acc_d.at[pl.ds(my_row0, ROWS_PER_TILE)])
        plsc.subcore_barrier()

        @pl.loop(0, NFULL)
        def _(b):
            off = base + b * BLK
            pltpu.sync_copy(src_hbm.at[pl.ds(off, BLK)], sidx)
            pltpu.sync_copy(dst_hbm.at[pl.ds(off, BLK)], didx)
            pltpu.sync_copy(ones, acc_s.at[sidx], add=True)
            pltpu.sync_copy(ones, acc_d.at[didx], add=True)

        toff = base + NFULL * BLK
        pltpu.sync_copy(src_hbm.at[pl.ds(toff, TAIL)], tsidx)
        pltpu.sync_copy(dst_hbm.at[pl.ds(toff, TAIL)], tdidx)
        pltpu.sync_copy(tones, acc_s.at[tsidx], add=True)
        pltpu.sync_copy(tones, acc_d.at[tdidx], add=True)

        plsc.subcore_barrier()

        pltpu.sync_copy(acc_s.at[pl.ds(my_row0, ROWS_PER_TILE)],
                        out_hbm.at[cid].at[0].at[pl.ds(my_row0, ROWS_PER_TILE)])
        pltpu.sync_copy(acc_d.at[pl.ds(my_row0, ROWS_PER_TILE)],
                        out_hbm.at[cid].at[1].at[pl.ds(my_row0, ROWS_PER_TILE)])

    return k(src, dst)


def _norms(d_ref):
    deg_out = d_ref[:, 0:1] + d_ref[:, 2:3]
    deg_in = d_ref[:, 1:2] + d_ref[:, 3:4]
    return (lax.rsqrt(jnp.maximum(deg_out, 1.0)),
            lax.rsqrt(jnp.maximum(deg_in, 1.0)))


def _tc_matmul(x, w):
    def body(x_ref, w_ref, o_ref):
        o_ref[...] = jnp.dot(x_ref[...], w_ref[...],
                             preferred_element_type=jnp.float32)
    return pl.pallas_call(
        body, out_shape=jax.ShapeDtypeStruct((x.shape[0], w.shape[1]), jnp.float32),
    )(x, w)


def _tc_scale_src(xw, degp_t):
    def body(xw_ref, d_ref, o_ref):
        ns, _ = _norms(d_ref)
        o_ref[...] = xw_ref[...] * ns
    return pl.pallas_call(
        body, out_shape=jax.ShapeDtypeStruct(xw.shape, jnp.float32),
    )(xw, degp_t)


def _tc_mid(partials, degp_t, b1, w2):
    def body(p_ref, d_ref, b_ref, w_ref, o_ref):
        ns, nd = _norms(d_ref)
        u = (p_ref[0, :N_NODES] + p_ref[1, :N_NODES]) * nd + b_ref[...]
        u = jnp.maximum(u, 0.0)
        o_ref[...] = jnp.dot(u, w_ref[...],
                             preferred_element_type=jnp.float32) * ns
    return pl.pallas_call(
        body, out_shape=jax.ShapeDtypeStruct((N_NODES, D), jnp.float32),
    )(partials, degp_t, b1, w2)


def _tc_final(partials, degp_t, b2):
    def body(p_ref, d_ref, b_ref, o_ref):
        _, nd = _norms(d_ref)
        o_ref[...] = (p_ref[0, :N_NODES] + p_ref[1, :N_NODES]) * nd + b_ref[...]
    return pl.pallas_call(
        body, out_shape=jax.ShapeDtypeStruct((N_NODES, D), jnp.float32),
    )(partials, degp_t, b2)


def kernel(features, edge_index, W1, b1, W2, b2):
    src = edge_index[0].astype(jnp.int32)
    dst = edge_index[1].astype(jnp.int32)
    b1r = b1.reshape(1, D)
    b2r = b2.reshape(1, D)

    degp = _sc_degrees(src, dst)
    degp_t = jnp.transpose(degp.reshape(NC * 2, N_PAD)[:, :N_NODES])

    xw1 = _tc_matmul(features, W1)
    h1 = _tc_scale_src(xw1, degp_t)
    p1 = _sc_message_pass(h1, src, dst)
    h2 = _tc_mid(p1, degp_t, b1r, W2)
    p2 = _sc_message_pass(h2, src, dst)
    return _tc_final(p2, degp_t, b2r)


# speedup vs baseline: 8.7755x; 1.3958x over previous
import functools

import jax
import jax.numpy as jnp
from jax import lax
from jax.experimental import pallas as pl
from jax.experimental.pallas import tpu as pltpu
from jax.experimental.pallas import tpu_sc as plsc

N_NODES = 10000
N_EDGES = 320000
D = 128

NC = 2
NS = 16
NW = NC * NS
EDGES_PER_W = N_EDGES // NW
BLK = 128
NFULL = EDGES_PER_W // BLK
TAIL = EDGES_PER_W - NFULL * BLK
N_PAD = 10240
ROWS_PER_TILE = N_PAD // NS
ZROWS = 128
PADE = N_PAD - EDGES_PER_W
NBLK = N_PAD // BLK

_mesh = lambda: plsc.VectorSubcoreMesh(core_axis_name="c", subcore_axis_name="s")


def _pad_edges(src, dst):
    srcw = src.reshape(NW, EDGES_PER_W)
    dstw = dst.reshape(NW, EDGES_PER_W)
    ar = jnp.arange(PADE, dtype=jnp.int32)
    wid = jnp.arange(NW, dtype=jnp.int32)[:, None]
    pad_s = jnp.broadcast_to((ar * 41) % N_NODES, (NW, PADE))
    pad_d = N_NODES + (ar[None, :] + wid * 7) % PADE
    src_p = jnp.concatenate([srcw, pad_s], axis=1).reshape(NW, NBLK, BLK)
    dst_p = jnp.concatenate([dstw, pad_d], axis=1).reshape(NW, NBLK, BLK)
    return src_p, dst_p


def _sc_message_pass(table, src_p, dst_p):
    @functools.partial(
        pl.kernel,
        out_type=jax.ShapeDtypeStruct((NC, N_PAD, D), jnp.float32),
        mesh=_mesh(),
        scratch_types=[
            pltpu.VMEM((BLK,), jnp.int32),
            pltpu.VMEM((BLK,), jnp.int32),
            pltpu.VMEM((BLK,), jnp.int32),
            pltpu.VMEM((BLK,), jnp.int32),
            pltpu.VMEM((BLK, D), jnp.float32),
            pltpu.VMEM((BLK, D), jnp.float32),
            pltpu.VMEM((ZROWS // 2, D), jnp.float32),
            pltpu.VMEM_SHARED((N_PAD, D), jnp.float32),
            pltpu.SemaphoreType.DMA,
            pltpu.SemaphoreType.DMA,
        ],
    )
    def k(table_hbm, srcp_hbm, dstp_hbm, out_hbm,
          sidx0, sidx1, didx0, didx1, rows0, rows1, zbuf, acc, gs0, gs1):
        cid = lax.axis_index("c")
        sid = lax.axis_index("s")
        wid = sid * NC + cid
        my_row0 = sid * ROWS_PER_TILE
        my_src = srcp_hbm.at[wid]
        my_dst = dstp_hbm.at[wid]

        @pl.loop(0, ZROWS // 2)
        def _(i):
            @pl.loop(0, D, step=16)
            def _(j):
                zbuf[i, pl.ds(j, 16)] = jnp.zeros((16,), jnp.float32)
        @pl.loop(0, ROWS_PER_TILE, step=ZROWS // 2)
        def _(r):
            pltpu.sync_copy(zbuf, acc.at[pl.ds(my_row0 + r, ZROWS // 2)])
        plsc.subcore_barrier()

        pltpu.sync_copy(my_src.at[0], sidx0)
        pltpu.sync_copy(my_dst.at[0], didx0)
        pltpu.async_copy(table_hbm.at[sidx0], rows0, gs0)

        @pl.loop(0, NBLK, step=2)
        def _(b):
            pltpu.sync_copy(my_src.at[b + 1], sidx1)
            pltpu.sync_copy(my_dst.at[b + 1], didx1)
            pltpu.make_async_copy(table_hbm.at[sidx0], rows0, gs0).wait()
            pltpu.async_copy(table_hbm.at[sidx1], rows1, gs1)
            pltpu.sync_copy(rows0, acc.at[didx0], add=True)

            @pl.when(b + 2 < NBLK)
            def _():
                pltpu.sync_copy(my_src.at[b + 2], sidx0)
                pltpu.sync_copy(my_dst.at[b + 2], didx0)

            pltpu.make_async_copy(table_hbm.at[sidx1], rows1, gs1).wait()

            @pl.when(b + 2 < NBLK)
            def _():
                pltpu.async_copy(table_hbm.at[sidx0], rows0, gs0)

            pltpu.sync_copy(rows1, acc.at[didx1], add=True)

        plsc.subcore_barrier()

        pltpu.sync_copy(acc.at[pl.ds(my_row0, ROWS_PER_TILE)],
                        out_hbm.at[cid].at[pl.ds(my_row0, ROWS_PER_TILE)])

    return k(table, src_p, dst_p)


def _sc_degrees(src, dst):
    @functools.partial(
        pl.kernel,
        out_type=jax.ShapeDtypeStruct((NC, 2, N_PAD), jnp.float32),
        mesh=_mesh(),
        scratch_types=[
            pltpu.VMEM((BLK,), jnp.int32),
            pltpu.VMEM((BLK,), jnp.int32),
            pltpu.VMEM((TAIL,), jnp.int32),
            pltpu.VMEM((TAIL,), jnp.int32),
            pltpu.VMEM((BLK,), jnp.float32),
            pltpu.VMEM((TAIL,), jnp.float32),
            pltpu.VMEM((ROWS_PER_TILE,), jnp.float32),
            pltpu.VMEM_SHARED((N_PAD,), jnp.float32),
            pltpu.VMEM_SHARED((N_PAD,), jnp.float32),
        ],
    )
    def k(src_hbm, dst_hbm, out_hbm,
          sidx, didx, tsidx, tdidx, ones, tones, zbuf, acc_s, acc_d):
        cid = lax.axis_index("c")
        sid = lax.axis_index("s")
        wid = sid * NC + cid
        base = wid * EDGES_PER_W
        my_row0 = sid * ROWS_PER_TILE

        @pl.loop(0, ROWS_PER_TILE, step=16)
        def _(j):
            zbuf[pl.ds(j, 16)] = jnp.zeros((16,), jnp.float32)
        @pl.loop(0, BLK, step=16)
        def _(j):
            ones[pl.ds(j, 16)] = jnp.ones((16,), jnp.float32)
        tones[pl.ds(0, TAIL)] = jnp.ones((TAIL,), jnp.float32)

        pltpu.sync_copy(zbuf, acc_s.at[pl.ds(my_row0, ROWS_PER_TILE)])
        pltpu.sync_copy(zbuf, acc_d.at[pl.ds(my_row0, ROWS_PER_TILE)])
        plsc.subcore_barrier()

        @pl.loop(0, NFULL)
        def _(b):
            off = base + b * BLK
            pltpu.sync_copy(src_hbm.at[pl.ds(off, BLK)], sidx)
            pltpu.sync_copy(dst_hbm.at[pl.ds(off, BLK)], didx)
            pltpu.sync_copy(ones, acc_s.at[sidx], add=True)
            pltpu.sync_copy(ones, acc_d.at[didx], add=True)

        toff = base + NFULL * BLK
        pltpu.sync_copy(src_hbm.at[pl.ds(toff, TAIL)], tsidx)
        pltpu.sync_copy(dst_hbm.at[pl.ds(toff, TAIL)], tdidx)
        pltpu.sync_copy(tones, acc_s.at[tsidx], add=True)
        pltpu.sync_copy(tones, acc_d.at[tdidx], add=True)

        plsc.subcore_barrier()

        pltpu.sync_copy(acc_s.at[pl.ds(my_row0, ROWS_PER_TILE)],
                        out_hbm.at[cid].at[0].at[pl.ds(my_row0, ROWS_PER_TILE)])
        pltpu.sync_copy(acc_d.at[pl.ds(my_row0, ROWS_PER_TILE)],
                        out_hbm.at[cid].at[1].at[pl.ds(my_row0, ROWS_PER_TILE)])

    return k(src, dst)


def _norms(d_ref):
    deg_out = d_ref[:, 0:1] + d_ref[:, 2:3]
    deg_in = d_ref[:, 1:2] + d_ref[:, 3:4]
    return (lax.rsqrt(jnp.maximum(deg_out, 1.0)),
            lax.rsqrt(jnp.maximum(deg_in, 1.0)))


def _tc_matmul(x, w):
    def body(x_ref, w_ref, o_ref):
        o_ref[...] = jnp.dot(x_ref[...], w_ref[...],
                             preferred_element_type=jnp.float32)
    return pl.pallas_call(
        body, out_shape=jax.ShapeDtypeStruct((x.shape[0], w.shape[1]), jnp.float32),
    )(x, w)


def _tc_scale_src(xw, degp_t):
    def body(xw_ref, d_ref, o_ref):
        ns, _ = _norms(d_ref)
        o_ref[...] = xw_ref[...] * ns
    return pl.pallas_call(
        body, out_shape=jax.ShapeDtypeStruct(xw.shape, jnp.float32),
    )(xw, degp_t)


def _tc_mid(partials, degp_t, b1, w2):
    def body(p_ref, d_ref, b_ref, w_ref, o_ref):
        ns, nd = _norms(d_ref)
        u = (p_ref[0, :N_NODES] + p_ref[1, :N_NODES]) * nd + b_ref[...]
        u = jnp.maximum(u, 0.0)
        o_ref[...] = jnp.dot(u, w_ref[...],
                             preferred_element_type=jnp.float32) * ns
    return pl.pallas_call(
        body, out_shape=jax.ShapeDtypeStruct((N_NODES, D), jnp.float32),
    )(partials, degp_t, b1, w2)


def _tc_final(partials, degp_t, b2):
    def body(p_ref, d_ref, b_ref, o_ref):
        _, nd = _norms(d_ref)
        o_ref[...] = (p_ref[0, :N_NODES] + p_ref[1, :N_NODES]) * nd + b_ref[...]
    return pl.pallas_call(
        body, out_shape=jax.ShapeDtypeStruct((N_NODES, D), jnp.float32),
    )(partials, degp_t, b2)


def kernel(features, edge_index, W1, b1, W2, b2):
    src = edge_index[0].astype(jnp.int32)
    dst = edge_index[1].astype(jnp.int32)
    b1r = b1.reshape(1, D)
    b2r = b2.reshape(1, D)

    degp = _sc_degrees(src, dst)
    degp_t = jnp.transpose(degp.reshape(NC * 2, N_PAD)[:, :N_NODES])

    src_p, dst_p = _pad_edges(src, dst)
    xw1 = _tc_matmul(features, W1)
    h1 = _tc_scale_src(xw1, degp_t)
    p1 = _sc_message_pass(h1, src_p, dst_p)
    h2 = _tc_mid(p1, degp_t, b1r, W2)
    p2 = _sc_message_pass(h2, src_p, dst_p)
    return _tc_final(p2, degp_t, b2r)


# speedup vs baseline: 10.5680x; 1.2043x over previous
import dataclasses
import functools

import jax
import jax.numpy as jnp
from jax import lax
from jax.experimental import pallas as pl
from jax.experimental.pallas import tpu as pltpu
from jax.experimental.pallas import tpu_sc as plsc

N_NODES = 10000
N_EDGES = 320000
D = 128

NC = 2
NS = 16
NW = NC * NS
EDGES_PER_W = N_EDGES // NW
BLK = 128
NFULL = EDGES_PER_W // BLK
TAIL = EDGES_PER_W - NFULL * BLK
N_PAD = 10240
ROWS_PER_TILE = N_PAD // NS
ZROWS = 128
PADE = N_PAD - EDGES_PER_W
NBLK = N_PAD // BLK

_mesh = lambda: plsc.VectorSubcoreMesh(core_axis_name="c", subcore_axis_name="s")


def _pad_edges(src, dst):
    srcw = src.reshape(NW, EDGES_PER_W)
    dstw = dst.reshape(NW, EDGES_PER_W)
    ar = jnp.arange(PADE, dtype=jnp.int32)
    wid = jnp.arange(NW, dtype=jnp.int32)[:, None]
    pad_s = jnp.broadcast_to((ar * 41) % N_NODES, (NW, PADE))
    pad_h = N_NODES + (ar[None, :] + wid * 7) % PADE
    src_p = jnp.concatenate([srcw, pad_s], axis=1).reshape(NW, NBLK, BLK)
    dst_p = jnp.concatenate([dstw, pad_h], axis=1).reshape(NW, NBLK, BLK)
    src_dp = jnp.concatenate([srcw, pad_h], axis=1).reshape(NW, NBLK, BLK)
    return src_p, dst_p, src_dp


def _sc_message_pass(table, src_p, dst_p):
    @functools.partial(
        pl.kernel,
        out_type=jax.ShapeDtypeStruct((NC, N_PAD, D), jnp.float32),
        mesh=_mesh(),
        scratch_types=[
            pltpu.VMEM((BLK,), jnp.int32),
            pltpu.VMEM((BLK,), jnp.int32),
            pltpu.VMEM((BLK,), jnp.int32),
            pltpu.VMEM((BLK,), jnp.int32),
            pltpu.VMEM((BLK, D), jnp.float32),
            pltpu.VMEM((BLK, D), jnp.float32),
            pltpu.VMEM((ZROWS // 2, D), jnp.float32),
            pltpu.VMEM_SHARED((N_PAD, D), jnp.float32),
            pltpu.SemaphoreType.DMA,
            pltpu.SemaphoreType.DMA,
        ],
    )
    def k(table_hbm, srcp_hbm, dstp_hbm, out_hbm,
          sidx0, sidx1, didx0, didx1, rows0, rows1, zbuf, acc, gs0, gs1):
        cid = lax.axis_index("c")
        sid = lax.axis_index("s")
        wid = sid * NC + cid
        my_row0 = sid * ROWS_PER_TILE
        my_src = srcp_hbm.at[wid]
        my_dst = dstp_hbm.at[wid]

        @pl.loop(0, ZROWS // 2)
        def _(i):
            @pl.loop(0, D, step=16)
            def _(j):
                zbuf[i, pl.ds(j, 16)] = jnp.zeros((16,), jnp.float32)
        @pl.loop(0, ROWS_PER_TILE, step=ZROWS // 2)
        def _(r):
            pltpu.sync_copy(zbuf, acc.at[pl.ds(my_row0 + r, ZROWS // 2)])
        plsc.subcore_barrier()

        pltpu.sync_copy(my_src.at[0], sidx0)
        pltpu.sync_copy(my_dst.at[0], didx0)
        pltpu.async_copy(table_hbm.at[sidx0], rows0, gs0)

        @pl.loop(0, NBLK, step=2)
        def _(b):
            pltpu.sync_copy(my_src.at[b + 1], sidx1)
            pltpu.sync_copy(my_dst.at[b + 1], didx1)
            pltpu.make_async_copy(table_hbm.at[sidx0], rows0, gs0).wait()
            pltpu.async_copy(table_hbm.at[sidx1], rows1, gs1)
            pltpu.sync_copy(rows0, acc.at[didx0], add=True)

            @pl.when(b + 2 < NBLK)
            def _():
                pltpu.sync_copy(my_src.at[b + 2], sidx0)
                pltpu.sync_copy(my_dst.at[b + 2], didx0)

            pltpu.make_async_copy(table_hbm.at[sidx1], rows1, gs1).wait()

            @pl.when(b + 2 < NBLK)
            def _():
                pltpu.async_copy(table_hbm.at[sidx0], rows0, gs0)

            pltpu.sync_copy(rows1, acc.at[didx1], add=True)

        plsc.subcore_barrier()

        pltpu.sync_copy(acc.at[pl.ds(my_row0, ROWS_PER_TILE)],
                        out_hbm.at[cid].at[pl.ds(my_row0, ROWS_PER_TILE)])

    return k(table, src_p, dst_p)


def _sc_degrees(src_dp, dst_p):
    cp = pltpu.CompilerParams()
    if "needs_layout_passes" in pltpu.CompilerParams.__dataclass_fields__:
        cp = dataclasses.replace(cp, needs_layout_passes=False)

    @functools.partial(
        pl.kernel,
        out_type=jax.ShapeDtypeStruct((2, NW, N_PAD), jnp.float32),
        mesh=_mesh(),
        compiler_params=cp,
        scratch_types=[
            pltpu.VMEM((NBLK, BLK), jnp.int32),
            pltpu.VMEM((NBLK, BLK), jnp.int32),
            pltpu.VMEM((N_PAD,), jnp.float32),
            pltpu.VMEM((N_PAD,), jnp.float32),
            pltpu.SemaphoreType.DMA,
        ],
    )
    def k(src_hbm, dst_hbm, out_hbm, sidx, didx, hist_s, hist_d, isem):
        cid = lax.axis_index("c")
        sid = lax.axis_index("s")
        wid = sid * NC + cid

        pltpu.async_copy(src_hbm.at[wid], sidx, isem)
        pltpu.async_copy(dst_hbm.at[wid], didx, isem)

        zeros = jnp.zeros((16,), jnp.float32)
        @pl.loop(0, N_PAD, step=16)
        def _(j):
            hist_s[pl.ds(j, 16)] = zeros
            hist_d[pl.ds(j, 16)] = zeros

        pltpu.make_async_copy(src_hbm.at[wid], sidx, isem).wait()
        pltpu.make_async_copy(dst_hbm.at[wid], didx, isem).wait()

        ones = jnp.ones((16,), jnp.float32)
        @pl.loop(0, NBLK)
        def _(b):
            @pl.loop(0, BLK, step=16)
            def _(j):
                plsc.addupdate_scatter(hist_s, [sidx[b, pl.ds(j, 16)]], ones)
                plsc.addupdate_scatter(hist_d, [didx[b, pl.ds(j, 16)]], ones)

        pltpu.sync_copy(hist_s, out_hbm.at[0].at[wid])
        pltpu.sync_copy(hist_d, out_hbm.at[1].at[wid])

    return k(src_dp, dst_p)


def _norms(d_ref):
    deg_out = jnp.sum(d_ref[:, :NW], axis=1, keepdims=True)
    deg_in = jnp.sum(d_ref[:, NW:], axis=1, keepdims=True)
    return (lax.rsqrt(jnp.maximum(deg_out, 1.0)),
            lax.rsqrt(jnp.maximum(deg_in, 1.0)))


def _tc_matmul(x, w):
    def body(x_ref, w_ref, o_ref):
        o_ref[...] = jnp.dot(x_ref[...], w_ref[...],
                             preferred_element_type=jnp.float32)
    return pl.pallas_call(
        body, out_shape=jax.ShapeDtypeStruct((x.shape[0], w.shape[1]), jnp.float32),
    )(x, w)


def _tc_scale_src(xw, degp_t):
    def body(xw_ref, d_ref, o_ref):
        ns, _ = _norms(d_ref)
        o_ref[...] = xw_ref[...] * ns
    return pl.pallas_call(
        body, out_shape=jax.ShapeDtypeStruct(xw.shape, jnp.float32),
    )(xw, degp_t)


def _tc_mid(partials, degp_t, b1, w2):
    def body(p_ref, d_ref, b_ref, w_ref, o_ref):
        ns, nd = _norms(d_ref)
        u = (p_ref[0, :N_NODES] + p_ref[1, :N_NODES]) * nd + b_ref[...]
        u = jnp.maximum(u, 0.0)
        o_ref[...] = jnp.dot(u, w_ref[...],
                             preferred_element_type=jnp.float32) * ns
    return pl.pallas_call(
        body, out_shape=jax.ShapeDtypeStruct((N_NODES, D), jnp.float32),
    )(partials, degp_t, b1, w2)


def _tc_final(partials, degp_t, b2):
    def body(p_ref, d_ref, b_ref, o_ref):
        _, nd = _norms(d_ref)
        o_ref[...] = (p_ref[0, :N_NODES] + p_ref[1, :N_NODES]) * nd + b_ref[...]
    return pl.pallas_call(
        body, out_shape=jax.ShapeDtypeStruct((N_NODES, D), jnp.float32),
    )(partials, degp_t, b2)


def kernel(features, edge_index, W1, b1, W2, b2):
    src = edge_index[0].astype(jnp.int32)
    dst = edge_index[1].astype(jnp.int32)
    b1r = b1.reshape(1, D)
    b2r = b2.reshape(1, D)

    src_p, dst_p, src_dp = _pad_edges(src, dst)
    degp = _sc_degrees(src_dp, dst_p)
    degp_t = jnp.transpose(degp.reshape(2 * NW, N_PAD)[:, :N_NODES])
    xw1 = _tc_matmul(features, W1)
    h1 = _tc_scale_src(xw1, degp_t)
    p1 = _sc_message_pass(h1, src_p, dst_p)
    h2 = _tc_mid(p1, degp_t, b1r, W2)
    p2 = _sc_message_pass(h2, src_p, dst_p)
    return _tc_final(p2, degp_t, b2r)


# speedup vs baseline: 11.8368x; 1.1201x over previous
import dataclasses
import functools

import jax
import jax.numpy as jnp
from jax import lax
from jax.experimental import pallas as pl
from jax.experimental.pallas import tpu as pltpu
from jax.experimental.pallas import tpu_sc as plsc

N_NODES = 10000
N_EDGES = 320000
D = 128

NC = 2
NS = 16
NW = NC * NS
EDGES_PER_W = N_EDGES // NW
BLK = 128
NFULL = EDGES_PER_W // BLK
TAIL = EDGES_PER_W - NFULL * BLK
N_PAD = 10240
ROWS_PER_TILE = N_PAD // NS
ZROWS = 128
PADE = N_PAD - EDGES_PER_W
NBLK = N_PAD // BLK

_mesh = lambda: plsc.VectorSubcoreMesh(core_axis_name="c", subcore_axis_name="s")


def _pad_edges(src, dst):
    srcw = src.reshape(NW, EDGES_PER_W)
    dstw = dst.reshape(NW, EDGES_PER_W)
    ar = jnp.arange(PADE, dtype=jnp.int32)
    wid = jnp.arange(NW, dtype=jnp.int32)[:, None]
    pad_s = jnp.broadcast_to((ar * 41) % N_NODES, (NW, PADE))
    pad_h = N_NODES + (ar[None, :] + wid * 7) % PADE
    src_p = jnp.concatenate([srcw, pad_s], axis=1).reshape(NW, NBLK, BLK)
    dst_p = jnp.concatenate([dstw, pad_h], axis=1).reshape(NW, NBLK, BLK)
    src_dp = jnp.concatenate([srcw, pad_h], axis=1).reshape(NW, NBLK, BLK)
    return src_p, dst_p, src_dp


def _sc_message_pass(table, src_p, dst_p):
    @functools.partial(
        pl.kernel,
        out_type=jax.ShapeDtypeStruct((NC, N_PAD, D), jnp.float32),
        mesh=_mesh(),
        scratch_types=[
            pltpu.VMEM((NBLK, BLK), jnp.int32),
            pltpu.VMEM((BLK,), jnp.int32),
            pltpu.VMEM((BLK,), jnp.int32),
            pltpu.VMEM((BLK, D), jnp.float32),
            pltpu.VMEM((BLK, D), jnp.float32),
            pltpu.VMEM((16, D), jnp.float32),
            pltpu.VMEM_SHARED((N_PAD, D), jnp.float32),
            pltpu.SemaphoreType.DMA,
            pltpu.SemaphoreType.DMA,
            pltpu.SemaphoreType.DMA,
            pltpu.SemaphoreType.DMA,
            pltpu.SemaphoreType.DMA,
        ],
    )
    def k(table_hbm, srcp_hbm, dstp_hbm, out_hbm,
          sidx, didx0, didx1, rows0, rows1, zbuf, acc, gs0, gs1, ss0, ss1, isem):
        cid = lax.axis_index("c")
        sid = lax.axis_index("s")
        wid = sid * NC + cid
        my_row0 = sid * ROWS_PER_TILE
        my_dst = dstp_hbm.at[wid]

        pltpu.async_copy(srcp_hbm.at[wid], sidx, isem)

        @pl.loop(0, 16)
        def _(i):
            @pl.loop(0, D, step=16)
            def _(j):
                zbuf[i, pl.ds(j, 16)] = jnp.zeros((16,), jnp.float32)
        @pl.loop(0, ROWS_PER_TILE, step=16)
        def _(r):
            pltpu.sync_copy(zbuf, acc.at[pl.ds(my_row0 + r, 16)])

        pltpu.make_async_copy(srcp_hbm.at[wid], sidx, isem).wait()
        plsc.subcore_barrier()

        pltpu.sync_copy(my_dst.at[0], didx0)
        pltpu.async_copy(table_hbm.at[sidx.at[0]], rows0, gs0)

        @pl.loop(0, NBLK, step=2)
        def _(b):
            @pl.when(b > 0)
            def _():
                pltpu.make_async_copy(rows1, acc.at[didx1], ss1).wait()
            pltpu.sync_copy(my_dst.at[b + 1], didx1)
            pltpu.make_async_copy(table_hbm.at[sidx.at[b]], rows0, gs0).wait()
            pltpu.async_copy(table_hbm.at[sidx.at[b + 1]], rows1, gs1)
            pltpu.make_async_copy(rows0, acc.at[didx0], ss0).start(add=True)

            @pl.when(b + 2 < NBLK)
            def _():
                pltpu.make_async_copy(rows0, acc.at[didx0], ss0).wait()
                pltpu.sync_copy(my_dst.at[b + 2], didx0)
            pltpu.make_async_copy(table_hbm.at[sidx.at[b + 1]], rows1, gs1).wait()
            @pl.when(b + 2 < NBLK)
            def _():
                pltpu.async_copy(table_hbm.at[sidx.at[b + 2]], rows0, gs0)
            pltpu.make_async_copy(rows1, acc.at[didx1], ss1).start(add=True)

        pltpu.make_async_copy(rows0, acc.at[didx0], ss0).wait()
        pltpu.make_async_copy(rows1, acc.at[didx1], ss1).wait()
        plsc.subcore_barrier()

        pltpu.sync_copy(acc.at[pl.ds(my_row0, ROWS_PER_TILE)],
                        out_hbm.at[cid].at[pl.ds(my_row0, ROWS_PER_TILE)])

    return k(table, src_p, dst_p)


def _sc_degrees(src_dp, dst_p):
    cp = pltpu.CompilerParams()
    if "needs_layout_passes" in pltpu.CompilerParams.__dataclass_fields__:
        cp = dataclasses.replace(cp, needs_layout_passes=False)

    @functools.partial(
        pl.kernel,
        out_type=jax.ShapeDtypeStruct((2, NW, N_PAD), jnp.float32),
        mesh=_mesh(),
        compiler_params=cp,
        scratch_types=[
            pltpu.VMEM((NBLK, BLK), jnp.int32),
            pltpu.VMEM((NBLK, BLK), jnp.int32),
            pltpu.VMEM((N_PAD,), jnp.float32),
            pltpu.VMEM((N_PAD,), jnp.float32),
            pltpu.SemaphoreType.DMA,
        ],
    )
    def k(src_hbm, dst_hbm, out_hbm, sidx, didx, hist_s, hist_d, isem):
        cid = lax.axis_index("c")
        sid = lax.axis_index("s")
        wid = sid * NC + cid

        pltpu.async_copy(src_hbm.at[wid], sidx, isem)
        pltpu.async_copy(dst_hbm.at[wid], didx, isem)

        zeros = jnp.zeros((16,), jnp.float32)
        @pl.loop(0, N_PAD, step=16)
        def _(j):
            hist_s[pl.ds(j, 16)] = zeros
            hist_d[pl.ds(j, 16)] = zeros

        pltpu.make_async_copy(src_hbm.at[wid], sidx, isem).wait()
        pltpu.make_async_copy(dst_hbm.at[wid], didx, isem).wait()

        ones = jnp.ones((16,), jnp.float32)
        @pl.loop(0, NBLK)
        def _(b):
            @pl.loop(0, BLK, step=16)
            def _(j):
                plsc.addupdate_scatter(hist_s, [sidx[b, pl.ds(j, 16)]], ones)
                plsc.addupdate_scatter(hist_d, [didx[b, pl.ds(j, 16)]], ones)

        pltpu.sync_copy(hist_s, out_hbm.at[0].at[wid])
        pltpu.sync_copy(hist_d, out_hbm.at[1].at[wid])

    return k(src_dp, dst_p)


def _norms(d_ref):
    deg_out = jnp.sum(d_ref[:, :NW], axis=1, keepdims=True)
    deg_in = jnp.sum(d_ref[:, NW:], axis=1, keepdims=True)
    return (lax.rsqrt(jnp.maximum(deg_out, 1.0)),
            lax.rsqrt(jnp.maximum(deg_in, 1.0)))


def _tc_matmul_scale(x, w, degp_t):
    def body(x_ref, w_ref, d_ref, o_ref):
        ns, _ = _norms(d_ref)
        o_ref[...] = jnp.dot(x_ref[...], w_ref[...],
                             preferred_element_type=jnp.float32) * ns
    return pl.pallas_call(
        body, out_shape=jax.ShapeDtypeStruct((x.shape[0], w.shape[1]), jnp.float32),
    )(x, w, degp_t)


def _tc_mid(partials, degp_t, b1, w2):
    def body(p_ref, d_ref, b_ref, w_ref, o_ref):
        ns, nd = _norms(d_ref)
        u = (p_ref[0, :N_NODES] + p_ref[1, :N_NODES]) * nd + b_ref[...]
        u = jnp.maximum(u, 0.0)
        o_ref[...] = jnp.dot(u, w_ref[...],
                             preferred_element_type=jnp.float32) * ns
    return pl.pallas_call(
        body, out_shape=jax.ShapeDtypeStruct((N_NODES, D), jnp.float32),
    )(partials, degp_t, b1, w2)


def _tc_final(partials, degp_t, b2):
    def body(p_ref, d_ref, b_ref, o_ref):
        _, nd = _norms(d_ref)
        o_ref[...] = (p_ref[0, :N_NODES] + p_ref[1, :N_NODES]) * nd + b_ref[...]
    return pl.pallas_call(
        body, out_shape=jax.ShapeDtypeStruct((N_NODES, D), jnp.float32),
    )(partials, degp_t, b2)


def kernel(features, edge_index, W1, b1, W2, b2):
    src = edge_index[0].astype(jnp.int32)
    dst = edge_index[1].astype(jnp.int32)
    b1r = b1.reshape(1, D)
    b2r = b2.reshape(1, D)

    src_p, dst_p, src_dp = _pad_edges(src, dst)
    degp = _sc_degrees(src_dp, dst_p)
    degp_t = jnp.transpose(degp.reshape(2 * NW, N_PAD)[:, :N_NODES])
    h1 = _tc_matmul_scale(features, W1, degp_t)
    p1 = _sc_message_pass(h1, src_p, dst_p)
    h2 = _tc_mid(p1, degp_t, b1r, W2)
    p2 = _sc_message_pass(h2, src_p, dst_p)
    return _tc_final(p2, degp_t, b2r)


# speedup vs baseline: 12.0427x; 1.0174x over previous
import dataclasses
import functools

import jax
import jax.numpy as jnp
from jax import lax
from jax.experimental import pallas as pl
from jax.experimental.pallas import tpu as pltpu
from jax.experimental.pallas import tpu_sc as plsc

N_NODES = 10000
N_EDGES = 320000
D = 128

NC = 2
NS = 16
NW = NC * NS
EDGES_PER_W = N_EDGES // NW
BLK = 128
NFULL = EDGES_PER_W // BLK
TAIL = EDGES_PER_W - NFULL * BLK
N_PAD = 10240
ROWS_PER_TILE = N_PAD // NS
ZROWS = 128
PADE = N_PAD - EDGES_PER_W
NBLK = N_PAD // BLK

_mesh = lambda: plsc.VectorSubcoreMesh(core_axis_name="c", subcore_axis_name="s")


def _pad_edges(src, dst):
    srcw = src.reshape(NW, EDGES_PER_W)
    dstw = dst.reshape(NW, EDGES_PER_W)
    ar = jnp.arange(PADE, dtype=jnp.int32)
    wid = jnp.arange(NW, dtype=jnp.int32)[:, None]
    pad_s = jnp.broadcast_to((ar * 41) % N_NODES, (NW, PADE))
    pad_h = N_NODES + (ar[None, :] + wid * 7) % PADE
    src_p = jnp.concatenate([srcw, pad_s], axis=1).reshape(NW, NBLK, BLK)
    dst_p = jnp.concatenate([dstw, pad_h], axis=1).reshape(NW, NBLK, BLK)
    src_dp = jnp.concatenate([srcw, pad_h], axis=1).reshape(NW, NBLK, BLK)
    return src_p, dst_p, src_dp


def _sc_message_pass(table, src_p, dst_p):
    @functools.partial(
        pl.kernel,
        out_type=jax.ShapeDtypeStruct((NC, N_PAD, D), jnp.float32),
        mesh=_mesh(),
        scratch_types=[
            pltpu.VMEM((NBLK, BLK), jnp.int32),
            pltpu.VMEM((BLK,), jnp.int32),
            pltpu.VMEM((BLK,), jnp.int32),
            pltpu.VMEM((BLK, D), jnp.float32),
            pltpu.VMEM((BLK, D), jnp.float32),
            pltpu.VMEM((40, D), jnp.float32),
            pltpu.VMEM_SHARED((N_PAD, D), jnp.float32),
            pltpu.SemaphoreType.DMA,
            pltpu.SemaphoreType.DMA,
            pltpu.SemaphoreType.DMA,
            pltpu.SemaphoreType.DMA,
            pltpu.SemaphoreType.DMA,
            pltpu.SemaphoreType.DMA,
        ],
    )
    def k(table_hbm, srcp_hbm, dstp_hbm, out_hbm,
          sidx, didx0, didx1, rows0, rows1, zbuf, acc, gs0, gs1, ss0, ss1,
          isem, zsem):
        cid = lax.axis_index("c")
        sid = lax.axis_index("s")
        wid = sid * NC + cid
        my_row0 = sid * ROWS_PER_TILE
        my_dst = dstp_hbm.at[wid]

        pltpu.async_copy(srcp_hbm.at[wid], sidx, isem)

        @pl.loop(0, 40)
        def _(i):
            @pl.loop(0, D, step=16)
            def _(j):
                zbuf[i, pl.ds(j, 16)] = jnp.zeros((16,), jnp.float32)
        @pl.loop(0, ROWS_PER_TILE, step=40)
        def _(r):
            pltpu.async_copy(zbuf, acc.at[pl.ds(my_row0 + r, 40)], zsem)

        pltpu.make_async_copy(srcp_hbm.at[wid], sidx, isem).wait()
        pltpu.sync_copy(my_dst.at[0], didx0)
        pltpu.async_copy(table_hbm.at[sidx.at[0]], rows0, gs0)

        @pl.loop(0, ROWS_PER_TILE, step=40)
        def _(r):
            pltpu.make_async_copy(zbuf, acc.at[pl.ds(my_row0 + r, 40)], zsem).wait()
        plsc.subcore_barrier()


        @pl.loop(0, NBLK, step=2)
        def _(b):
            @pl.when(b > 0)
            def _():
                pltpu.make_async_copy(rows1, acc.at[didx1], ss1).wait()
            pltpu.sync_copy(my_dst.at[b + 1], didx1)
            pltpu.make_async_copy(table_hbm.at[sidx.at[b]], rows0, gs0).wait()
            pltpu.async_copy(table_hbm.at[sidx.at[b + 1]], rows1, gs1)
            pltpu.make_async_copy(rows0, acc.at[didx0], ss0).start(add=True)

            @pl.when(b + 2 < NBLK)
            def _():
                pltpu.make_async_copy(rows0, acc.at[didx0], ss0).wait()
                pltpu.sync_copy(my_dst.at[b + 2], didx0)
            pltpu.make_async_copy(table_hbm.at[sidx.at[b + 1]], rows1, gs1).wait()
            @pl.when(b + 2 < NBLK)
            def _():
                pltpu.async_copy(table_hbm.at[sidx.at[b + 2]], rows0, gs0)
            pltpu.make_async_copy(rows1, acc.at[didx1], ss1).start(add=True)

        pltpu.make_async_copy(rows0, acc.at[didx0], ss0).wait()
        pltpu.make_async_copy(rows1, acc.at[didx1], ss1).wait()
        plsc.subcore_barrier()

        pltpu.sync_copy(acc.at[pl.ds(my_row0, ROWS_PER_TILE)],
                        out_hbm.at[cid].at[pl.ds(my_row0, ROWS_PER_TILE)])

    return k(table, src_p, dst_p)


def _sc_degrees(src_dp, dst_p):
    cp = pltpu.CompilerParams()
    if "needs_layout_passes" in pltpu.CompilerParams.__dataclass_fields__:
        cp = dataclasses.replace(cp, needs_layout_passes=False)

    @functools.partial(
        pl.kernel,
        out_type=jax.ShapeDtypeStruct((2, NW, N_PAD), jnp.float32),
        mesh=_mesh(),
        compiler_params=cp,
        scratch_types=[
            pltpu.VMEM((NBLK, BLK), jnp.int32),
            pltpu.VMEM((NBLK, BLK), jnp.int32),
            pltpu.VMEM((N_PAD,), jnp.float32),
            pltpu.VMEM((N_PAD,), jnp.float32),
            pltpu.SemaphoreType.DMA,
        ],
    )
    def k(src_hbm, dst_hbm, out_hbm, sidx, didx, hist_s, hist_d, isem):
        cid = lax.axis_index("c")
        sid = lax.axis_index("s")
        wid = sid * NC + cid

        pltpu.async_copy(src_hbm.at[wid], sidx, isem)
        pltpu.async_copy(dst_hbm.at[wid], didx, isem)

        zeros = jnp.zeros((16,), jnp.float32)
        @pl.loop(0, N_PAD, step=16)
        def _(j):
            hist_s[pl.ds(j, 16)] = zeros
            hist_d[pl.ds(j, 16)] = zeros

        pltpu.make_async_copy(src_hbm.at[wid], sidx, isem).wait()
        pltpu.make_async_copy(dst_hbm.at[wid], didx, isem).wait()

        ones = jnp.ones((16,), jnp.float32)
        @pl.loop(0, NBLK)
        def _(b):
            @pl.loop(0, BLK, step=16)
            def _(j):
                plsc.addupdate_scatter(hist_s, [sidx[b, pl.ds(j, 16)]], ones)
                plsc.addupdate_scatter(hist_d, [didx[b, pl.ds(j, 16)]], ones)

        pltpu.sync_copy(hist_s, out_hbm.at[0].at[wid])
        pltpu.sync_copy(hist_d, out_hbm.at[1].at[wid])

    return k(src_dp, dst_p)


def _norms(d_ref):
    deg_out = jnp.sum(d_ref[:, :NW], axis=1, keepdims=True)
    deg_in = jnp.sum(d_ref[:, NW:], axis=1, keepdims=True)
    return (lax.rsqrt(jnp.maximum(deg_out, 1.0)),
            lax.rsqrt(jnp.maximum(deg_in, 1.0)))


def _tc_matmul_scale(x, w, degp_t):
    def body(x_ref, w_ref, d_ref, o_ref):
        ns, _ = _norms(d_ref)
        o_ref[...] = jnp.dot(x_ref[...], w_ref[...],
                             preferred_element_type=jnp.float32) * ns
    return pl.pallas_call(
        body, out_shape=jax.ShapeDtypeStruct((x.shape[0], w.shape[1]), jnp.float32),
    )(x, w, degp_t)


def _tc_mid(partials, degp_t, b1, w2):
    def body(p_ref, d_ref, b_ref, w_ref, o_ref):
        ns, nd = _norms(d_ref)
        u = (p_ref[0, :N_NODES] + p_ref[1, :N_NODES]) * nd + b_ref[...]
        u = jnp.maximum(u, 0.0)
        o_ref[...] = jnp.dot(u, w_ref[...],
                             preferred_element_type=jnp.float32) * ns
    return pl.pallas_call(
        body, out_shape=jax.ShapeDtypeStruct((N_NODES, D), jnp.float32),
    )(partials, degp_t, b1, w2)


def _tc_final(partials, degp_t, b2):
    def body(p_ref, d_ref, b_ref, o_ref):
        _, nd = _norms(d_ref)
        o_ref[...] = (p_ref[0, :N_NODES] + p_ref[1, :N_NODES]) * nd + b_ref[...]
    return pl.pallas_call(
        body, out_shape=jax.ShapeDtypeStruct((N_NODES, D), jnp.float32),
    )(partials, degp_t, b2)


def kernel(features, edge_index, W1, b1, W2, b2):
    src = edge_index[0].astype(jnp.int32)
    dst = edge_index[1].astype(jnp.int32)
    b1r = b1.reshape(1, D)
    b2r = b2.reshape(1, D)

    src_p, dst_p, src_dp = _pad_edges(src, dst)
    degp = _sc_degrees(src_dp, dst_p)
    degp_t = jnp.transpose(degp.reshape(2 * NW, N_PAD)[:, :N_NODES])
    h1 = _tc_matmul_scale(features, W1, degp_t)
    p1 = _sc_message_pass(h1, src_p, dst_p)
    h2 = _tc_mid(p1, degp_t, b1r, W2)
    p2 = _sc_message_pass(h2, src_p, dst_p)
    return _tc_final(p2, degp_t, b2r)


# speedup vs baseline: 13.5687x; 1.1267x over previous
import dataclasses
import functools

import jax
import jax.numpy as jnp
from jax import lax
from jax.experimental import pallas as pl
from jax.experimental.pallas import tpu as pltpu
from jax.experimental.pallas import tpu_sc as plsc

N_NODES = 10000
N_EDGES = 320000
D = 128

NC = 2
NS = 16
NW = NC * NS
EDGES_PER_W = N_EDGES // NW
BLK = 64
N_PAD = 10240
ROWS_PER_TILE = N_PAD // NS
PADE = N_PAD - EDGES_PER_W
NBLK = N_PAD // BLK

_mesh = lambda: plsc.VectorSubcoreMesh(core_axis_name="c", subcore_axis_name="s")


def _pad_edges(src, dst):
    srcw = src.reshape(NW, EDGES_PER_W)
    dstw = dst.reshape(NW, EDGES_PER_W)
    ar = jnp.arange(PADE, dtype=jnp.int32)
    wid = jnp.arange(NW, dtype=jnp.int32)[:, None]
    pad_s = jnp.broadcast_to((ar * 41) % N_NODES, (NW, PADE))
    pad_h = N_NODES + (ar[None, :] + wid * 7) % PADE
    src_p = jnp.concatenate([srcw, pad_s], axis=1)
    dst_p = jnp.concatenate([dstw, pad_h], axis=1).reshape(NW, NBLK, BLK)
    src_dp = jnp.concatenate([srcw, pad_h], axis=1).reshape(NW, NBLK, BLK)
    return src_p, dst_p, src_dp


def _sc_message_pass(table, src_p, dst_p):
    @functools.partial(
        pl.kernel,
        out_type=jax.ShapeDtypeStruct((NC, N_PAD, D), jnp.float32),
        mesh=_mesh(),
        scratch_types=[
            pltpu.VMEM((N_PAD,), jnp.int32),
            pltpu.VMEM((BLK,), jnp.int32),
            pltpu.VMEM((BLK,), jnp.int32),
            pltpu.VMEM((BLK,), jnp.int32),
            pltpu.VMEM((BLK,), jnp.int32),
            pltpu.VMEM((BLK, D), jnp.float32),
            pltpu.VMEM((BLK, D), jnp.float32),
            pltpu.VMEM((BLK, D), jnp.float32),
            pltpu.VMEM((BLK, D), jnp.float32),
            pltpu.VMEM((32, D), jnp.float32),
            pltpu.VMEM_SHARED((N_PAD, D), jnp.float32),
            pltpu.SemaphoreType.DMA,
            pltpu.SemaphoreType.DMA,
            pltpu.SemaphoreType.DMA,
            pltpu.SemaphoreType.DMA,
            pltpu.SemaphoreType.DMA,
            pltpu.SemaphoreType.DMA,
            pltpu.SemaphoreType.DMA,
            pltpu.SemaphoreType.DMA,
            pltpu.SemaphoreType.DMA,
            pltpu.SemaphoreType.DMA,
        ],
    )
    def k(table_hbm, srcp_hbm, dstp_hbm, out_hbm,
          sidx, didx0, didx1, didx2, didx3, rows0, rows1, rows2, rows3,
          zbuf, acc, gs0, gs1, gs2, gs3, ss0, ss1, ss2, ss3, isem, zsem):
        cid = lax.axis_index("c")
        sid = lax.axis_index("s")
        wid = sid * NC + cid
        my_row0 = sid * ROWS_PER_TILE
        my_dst = dstp_hbm.at[wid]
        didx = [didx0, didx1, didx2, didx3]
        rows = [rows0, rows1, rows2, rows3]
        gs = [gs0, gs1, gs2, gs3]
        ss = [ss0, ss1, ss2, ss3]

        pltpu.async_copy(srcp_hbm.at[wid], sidx, isem)

        @pl.loop(0, 32)
        def _(i):
            @pl.loop(0, D, step=16)
            def _(j):
                zbuf[i, pl.ds(j, 16)] = jnp.zeros((16,), jnp.float32)
        @pl.loop(0, ROWS_PER_TILE, step=32)
        def _(r):
            pltpu.async_copy(zbuf, acc.at[pl.ds(my_row0 + r, 32)], zsem)

        pltpu.make_async_copy(srcp_hbm.at[wid], sidx, isem).wait()
        pltpu.sync_copy(my_dst.at[0], didx0)
        pltpu.sync_copy(my_dst.at[1], didx1)
        pltpu.async_copy(table_hbm.at[sidx.at[pl.ds(0, BLK)]], rows0, gs0)
        pltpu.async_copy(table_hbm.at[sidx.at[pl.ds(BLK, BLK)]], rows1, gs1)

        @pl.loop(0, ROWS_PER_TILE, step=32)
        def _(r):
            pltpu.make_async_copy(zbuf, acc.at[pl.ds(my_row0 + r, 32)], zsem).wait()
        plsc.subcore_barrier()

        @pl.loop(0, NBLK, step=4)
        def _(b):
            for kk in range(4):
                bk = b + kk
                j = (kk + 2) % 4

                @pl.when(bk >= 2)
                def _():
                    pltpu.make_async_copy(rows[j], acc.at[didx[j]], ss[j]).wait()

                @pl.when(bk + 2 < NBLK)
                def _():
                    pltpu.sync_copy(my_dst.at[bk + 2], didx[j])

                pltpu.make_async_copy(table_hbm.at[sidx.at[pl.ds(bk * BLK, BLK)]],
                                      rows[kk], gs[kk]).wait()

                @pl.when(bk + 2 < NBLK)
                def _():
                    pltpu.async_copy(
                        table_hbm.at[sidx.at[pl.ds((bk + 2) * BLK, BLK)]],
                        rows[j], gs[j])

                pltpu.make_async_copy(rows[kk], acc.at[didx[kk]],
                                      ss[kk]).start(add=True)

        pltpu.make_async_copy(rows2, acc.at[didx2], ss2).wait()
        pltpu.make_async_copy(rows3, acc.at[didx3], ss3).wait()
        plsc.subcore_barrier()

        pltpu.sync_copy(acc.at[pl.ds(my_row0, ROWS_PER_TILE)],
                        out_hbm.at[cid].at[pl.ds(my_row0, ROWS_PER_TILE)])

    return k(table, src_p, dst_p)


def _sc_degrees(src_dp, dst_p):
    cp = pltpu.CompilerParams()
    if "needs_layout_passes" in pltpu.CompilerParams.__dataclass_fields__:
        cp = dataclasses.replace(cp, needs_layout_passes=False)

    @functools.partial(
        pl.kernel,
        out_type=jax.ShapeDtypeStruct((2, NW, N_PAD), jnp.float32),
        mesh=_mesh(),
        compiler_params=cp,
        scratch_types=[
            pltpu.VMEM((NBLK, BLK), jnp.int32),
            pltpu.VMEM((NBLK, BLK), jnp.int32),
            pltpu.VMEM((N_PAD,), jnp.float32),
            pltpu.VMEM((N_PAD,), jnp.float32),
            pltpu.SemaphoreType.DMA,
        ],
    )
    def k(src_hbm, dst_hbm, out_hbm, sidx, didx, hist_s, hist_d, isem):
        cid = lax.axis_index("c")
        sid = lax.axis_index("s")
        wid = sid * NC + cid

        pltpu.async_copy(src_hbm.at[wid], sidx, isem)
        pltpu.async_copy(dst_hbm.at[wid], didx, isem)

        zeros = jnp.zeros((16,), jnp.float32)
        @pl.loop(0, N_PAD, step=16)
        def _(j):
            hist_s[pl.ds(j, 16)] = zeros
            hist_d[pl.ds(j, 16)] = zeros

        pltpu.make_async_copy(src_hbm.at[wid], sidx, isem).wait()
        pltpu.make_async_copy(dst_hbm.at[wid], didx, isem).wait()

        ones = jnp.ones((16,), jnp.float32)
        @pl.loop(0, NBLK)
        def _(b):
            @pl.loop(0, BLK, step=16)
            def _(j):
                plsc.addupdate_scatter(hist_s, [sidx[b, pl.ds(j, 16)]], ones)
                plsc.addupdate_scatter(hist_d, [didx[b, pl.ds(j, 16)]], ones)

        pltpu.sync_copy(hist_s, out_hbm.at[0].at[wid])
        pltpu.sync_copy(hist_d, out_hbm.at[1].at[wid])

    return k(src_dp, dst_p)


def _norms(d_ref):
    deg_out = jnp.sum(d_ref[:, :NW], axis=1, keepdims=True)
    deg_in = jnp.sum(d_ref[:, NW:], axis=1, keepdims=True)
    return (lax.rsqrt(jnp.maximum(deg_out, 1.0)),
            lax.rsqrt(jnp.maximum(deg_in, 1.0)))


def _tc_matmul_scale(x, w, degp_t):
    def body(x_ref, w_ref, d_ref, o_ref):
        ns, _ = _norms(d_ref)
        o_ref[...] = jnp.dot(x_ref[...], w_ref[...],
                             preferred_element_type=jnp.float32) * ns
    return pl.pallas_call(
        body, out_shape=jax.ShapeDtypeStruct((x.shape[0], w.shape[1]), jnp.float32),
    )(x, w, degp_t)


def _tc_mid(partials, degp_t, b1, w2):
    def body(p_ref, d_ref, b_ref, w_ref, o_ref):
        ns, nd = _norms(d_ref)
        u = (p_ref[0, :N_NODES] + p_ref[1, :N_NODES]) * nd + b_ref[...]
        u = jnp.maximum(u, 0.0)
        o_ref[...] = jnp.dot(u, w_ref[...],
                             preferred_element_type=jnp.float32) * ns
    return pl.pallas_call(
        body, out_shape=jax.ShapeDtypeStruct((N_NODES, D), jnp.float32),
    )(partials, degp_t, b1, w2)


def _tc_final(partials, degp_t, b2):
    def body(p_ref, d_ref, b_ref, o_ref):
        _, nd = _norms(d_ref)
        o_ref[...] = (p_ref[0, :N_NODES] + p_ref[1, :N_NODES]) * nd + b_ref[...]
    return pl.pallas_call(
        body, out_shape=jax.ShapeDtypeStruct((N_NODES, D), jnp.float32),
    )(partials, degp_t, b2)


def kernel(features, edge_index, W1, b1, W2, b2):
    src = edge_index[0].astype(jnp.int32)
    dst = edge_index[1].astype(jnp.int32)
    b1r = b1.reshape(1, D)
    b2r = b2.reshape(1, D)

    src_p, dst_p, src_dp = _pad_edges(src, dst)
    degp = _sc_degrees(src_dp, dst_p)
    degp_t = jnp.transpose(degp.reshape(2 * NW, N_PAD)[:, :N_NODES])
    h1 = _tc_matmul_scale(features, W1, degp_t)
    p1 = _sc_message_pass(h1, src_p, dst_p)
    h2 = _tc_mid(p1, degp_t, b1r, W2)
    p2 = _sc_message_pass(h2, src_p, dst_p)
    return _tc_final(p2, degp_t, b2r)


# speedup vs baseline: 13.7933x; 1.0166x over previous
import dataclasses
import functools

import jax
import jax.numpy as jnp
from jax import lax
from jax.experimental import pallas as pl
from jax.experimental.pallas import tpu as pltpu
from jax.experimental.pallas import tpu_sc as plsc

N_NODES = 10000
N_EDGES = 320000
D = 128

NC = 2
NS = 16
NW = NC * NS
EDGES_PER_W = N_EDGES // NW
BLK = 64
N_PAD = 10240
ROWS_PER_TILE = N_PAD // NS
PADE = N_PAD - EDGES_PER_W
NBLK = N_PAD // BLK

_mesh = lambda: plsc.VectorSubcoreMesh(core_axis_name="c", subcore_axis_name="s")


def _pad_edges(src, dst):
    srcw = src.reshape(NW, EDGES_PER_W)
    dstw = dst.reshape(NW, EDGES_PER_W)
    ar = jnp.arange(PADE, dtype=jnp.int32)
    wid = jnp.arange(NW, dtype=jnp.int32)[:, None]
    pad_s = jnp.broadcast_to((ar * 41) % N_NODES, (NW, PADE))
    pad_h = N_NODES + (ar[None, :] + wid * 7) % PADE
    src_p = jnp.concatenate([srcw, pad_s], axis=1)
    dst_p = jnp.concatenate([dstw, pad_h], axis=1).reshape(NW, NBLK, BLK)
    return src_p, dst_p


def _sc_message_pass(table, src_p, dst_p):
    @functools.partial(
        pl.kernel,
        out_type=jax.ShapeDtypeStruct((NC, N_PAD, D), jnp.float32),
        mesh=_mesh(),
        scratch_types=[
            pltpu.VMEM((N_PAD,), jnp.int32),
            pltpu.VMEM((BLK,), jnp.int32),
            pltpu.VMEM((BLK,), jnp.int32),
            pltpu.VMEM((BLK,), jnp.int32),
            pltpu.VMEM((BLK,), jnp.int32),
            pltpu.VMEM((BLK, D), jnp.float32),
            pltpu.VMEM((BLK, D), jnp.float32),
            pltpu.VMEM((BLK, D), jnp.float32),
            pltpu.VMEM((BLK, D), jnp.float32),
            pltpu.VMEM((32, D), jnp.float32),
            pltpu.VMEM_SHARED((N_PAD, D), jnp.float32),
            pltpu.SemaphoreType.DMA,
            pltpu.SemaphoreType.DMA,
            pltpu.SemaphoreType.DMA,
            pltpu.SemaphoreType.DMA,
            pltpu.SemaphoreType.DMA,
            pltpu.SemaphoreType.DMA,
            pltpu.SemaphoreType.DMA,
            pltpu.SemaphoreType.DMA,
            pltpu.SemaphoreType.DMA,
            pltpu.SemaphoreType.DMA,
        ],
    )
    def k(table_hbm, srcp_hbm, dstp_hbm, out_hbm,
          sidx, didx0, didx1, didx2, didx3, rows0, rows1, rows2, rows3,
          zbuf, acc, gs0, gs1, gs2, gs3, ss0, ss1, ss2, ss3, isem, zsem):
        cid = lax.axis_index("c")
        sid = lax.axis_index("s")
        wid = sid * NC + cid
        my_row0 = sid * ROWS_PER_TILE
        my_dst = dstp_hbm.at[wid]
        didx = [didx0, didx1, didx2, didx3]
        rows = [rows0, rows1, rows2, rows3]
        gs = [gs0, gs1, gs2, gs3]
        ss = [ss0, ss1, ss2, ss3]

        pltpu.async_copy(srcp_hbm.at[wid], sidx, isem)

        @pl.loop(0, 32)
        def _(i):
            @pl.loop(0, D, step=16)
            def _(j):
                zbuf[i, pl.ds(j, 16)] = jnp.zeros((16,), jnp.float32)
        @pl.loop(0, ROWS_PER_TILE, step=32)
        def _(r):
            pltpu.async_copy(zbuf, acc.at[pl.ds(my_row0 + r, 32)], zsem)

        pltpu.make_async_copy(srcp_hbm.at[wid], sidx, isem).wait()
        pltpu.sync_copy(my_dst.at[0], didx0)
        pltpu.sync_copy(my_dst.at[1], didx1)
        pltpu.async_copy(table_hbm.at[sidx.at[pl.ds(0, BLK)]], rows0, gs0)
        pltpu.async_copy(table_hbm.at[sidx.at[pl.ds(BLK, BLK)]], rows1, gs1)

        @pl.loop(0, ROWS_PER_TILE, step=32)
        def _(r):
            pltpu.make_async_copy(zbuf, acc.at[pl.ds(my_row0 + r, 32)], zsem).wait()
        plsc.subcore_barrier()

        @pl.loop(0, NBLK, step=4)
        def _(b):
            for kk in range(4):
                bk = b + kk
                j = (kk + 2) % 4

                @pl.when(bk >= 2)
                def _():
                    pltpu.make_async_copy(rows[j], acc.at[didx[j]], ss[j]).wait()

                @pl.when(bk + 2 < NBLK)
                def _():
                    pltpu.sync_copy(my_dst.at[bk + 2], didx[j])

                pltpu.make_async_copy(table_hbm.at[sidx.at[pl.ds(bk * BLK, BLK)]],
                                      rows[kk], gs[kk]).wait()

                @pl.when(bk + 2 < NBLK)
                def _():
                    pltpu.async_copy(
                        table_hbm.at[sidx.at[pl.ds((bk + 2) * BLK, BLK)]],
                        rows[j], gs[j])

                pltpu.make_async_copy(rows[kk], acc.at[didx[kk]],
                                      ss[kk]).start(add=True)

        pltpu.make_async_copy(rows2, acc.at[didx2], ss2).wait()
        pltpu.make_async_copy(rows3, acc.at[didx3], ss3).wait()
        plsc.subcore_barrier()

        pltpu.sync_copy(acc.at[pl.ds(my_row0, ROWS_PER_TILE)],
                        out_hbm.at[cid].at[pl.ds(my_row0, ROWS_PER_TILE)])

    return k(table, src_p, dst_p)


def _sc_degrees(srcw, dstw):
    cp = pltpu.CompilerParams()
    if "needs_layout_passes" in pltpu.CompilerParams.__dataclass_fields__:
        cp = dataclasses.replace(cp, needs_layout_passes=False)

    @functools.partial(
        pl.kernel,
        out_type=jax.ShapeDtypeStruct((2, NW, N_PAD), jnp.float32),
        mesh=_mesh(),
        compiler_params=cp,
        scratch_types=[
            pltpu.VMEM((EDGES_PER_W,), jnp.int32),
            pltpu.VMEM((EDGES_PER_W,), jnp.int32),
            pltpu.VMEM((N_PAD,), jnp.float32),
            pltpu.VMEM((N_PAD,), jnp.float32),
            pltpu.SemaphoreType.DMA,
        ],
    )
    def k(src_hbm, dst_hbm, out_hbm, sidx, didx, hist_s, hist_d, isem):
        cid = lax.axis_index("c")
        sid = lax.axis_index("s")
        wid = sid * NC + cid

        pltpu.async_copy(src_hbm.at[wid], sidx, isem)
        pltpu.async_copy(dst_hbm.at[wid], didx, isem)

        zeros = jnp.zeros((16,), jnp.float32)
        @pl.loop(0, N_PAD, step=16)
        def _(j):
            hist_s[pl.ds(j, 16)] = zeros
            hist_d[pl.ds(j, 16)] = zeros

        pltpu.make_async_copy(src_hbm.at[wid], sidx, isem).wait()
        pltpu.make_async_copy(dst_hbm.at[wid], didx, isem).wait()

        ones = jnp.ones((16,), jnp.float32)
        @pl.loop(0, EDGES_PER_W, step=16)
        def _(e):
            plsc.addupdate_scatter(hist_s, [sidx[pl.ds(e, 16)]], ones)
            plsc.addupdate_scatter(hist_d, [didx[pl.ds(e, 16)]], ones)

        pltpu.sync_copy(hist_s, out_hbm.at[0].at[wid])
        pltpu.sync_copy(hist_d, out_hbm.at[1].at[wid])

    return k(srcw, dstw)


def _norms(d_ref):
    deg_out = jnp.sum(d_ref[:, :NW], axis=1, keepdims=True)
    deg_in = jnp.sum(d_ref[:, NW:], axis=1, keepdims=True)
    return (lax.rsqrt(jnp.maximum(deg_out, 1.0)),
            lax.rsqrt(jnp.maximum(deg_in, 1.0)))


def _tc_matmul_scale(x, w, degp_t):
    def body(x_ref, w_ref, d_ref, o_ref):
        ns, _ = _norms(d_ref)
        o_ref[...] = jnp.dot(x_ref[...], w_ref[...],
                             preferred_element_type=jnp.float32) * ns
    return pl.pallas_call(
        body, out_shape=jax.ShapeDtypeStruct((x.shape[0], w.shape[1]), jnp.float32),
    )(x, w, degp_t)


def _tc_mid(partials, degp_t, b1, w2):
    def body(p_ref, d_ref, b_ref, w_ref, o_ref):
        ns, nd = _norms(d_ref)
        u = (p_ref[0, :N_NODES] + p_ref[1, :N_NODES]) * nd + b_ref[...]
        u = jnp.maximum(u, 0.0)
        o_ref[...] = jnp.dot(u, w_ref[...],
                             preferred_element_type=jnp.float32) * ns
    return pl.pallas_call(
        body, out_shape=jax.ShapeDtypeStruct((N_NODES, D), jnp.float32),
    )(partials, degp_t, b1, w2)


def _tc_final(partials, degp_t, b2):
    def body(p_ref, d_ref, b_ref, o_ref):
        _, nd = _norms(d_ref)
        o_ref[...] = (p_ref[0, :N_NODES] + p_ref[1, :N_NODES]) * nd + b_ref[...]
    return pl.pallas_call(
        body, out_shape=jax.ShapeDtypeStruct((N_NODES, D), jnp.float32),
    )(partials, degp_t, b2)


def kernel(features, edge_index, W1, b1, W2, b2):
    src = edge_index[0].astype(jnp.int32)
    dst = edge_index[1].astype(jnp.int32)
    b1r = b1.reshape(1, D)
    b2r = b2.reshape(1, D)

    src_p, dst_p = _pad_edges(src, dst)
    degp = _sc_degrees(src.reshape(NW, EDGES_PER_W),
                       dst.reshape(NW, EDGES_PER_W))
    degp_t = jnp.transpose(degp.reshape(2 * NW, N_PAD)[:, :N_NODES])
    h1 = _tc_matmul_scale(features, W1, degp_t)
    p1 = _sc_message_pass(h1, src_p, dst_p)
    h2 = _tc_mid(p1, degp_t, b1r, W2)
    p2 = _sc_message_pass(h2, src_p, dst_p)
    return _tc_final(p2, degp_t, b2r)
